# Initial kernel scaffold; baseline (speedup 1.0000x reference)
import jax
import jax.numpy as jnp
from jax.experimental import pallas as pl
from jax.experimental.pallas import tpu as pltpu

S, D, FF, E = 2048, 768, 3072, 8
FFB = 768
NF = FF // FFB
_SQRT2 = 1.4142135623730951


def _moe_dense_body(x_ref, wr_ref, w1_ref, b1_ref, w2_ref, b2_ref, out_ref,
                    gates_ref):
    e = pl.program_id(0)
    f = pl.program_id(1)

    @pl.when((e == 0) & (f == 0))
    def _init():
        logits = jnp.dot(x_ref[...], wr_ref[...],
                         preferred_element_type=jnp.float32)
        m = jnp.max(logits, axis=-1, keepdims=True)
        p = jnp.exp(logits - m)
        p = p / jnp.sum(p, axis=-1, keepdims=True)
        p1 = jnp.max(p, axis=-1, keepdims=True)
        is1 = p == p1
        pm = jnp.where(is1, -1.0, p)
        p2 = jnp.max(pm, axis=-1, keepdims=True)
        is2 = pm == p2
        denom = p1 + p2
        gates_ref[...] = jnp.where(is1, p1 / denom,
                                   jnp.where(is2, p2 / denom, 0.0))
        out_ref[...] = jnp.zeros_like(out_ref)

    g = jax.lax.dynamic_slice(gates_ref[...], (0, e), (S, 1))
    h = jnp.dot(x_ref[...], w1_ref[0], preferred_element_type=jnp.float32)
    h = h + b1_ref[0][None, :]
    h = 0.5 * h * (1.0 + jax.lax.erf(h / _SQRT2))
    y = jnp.dot(h, w2_ref[0], preferred_element_type=jnp.float32)
    out_ref[...] += g * y

    @pl.when(f == 0)
    def _bias2():
        out_ref[...] += g * b2_ref[0][None, :]


def kernel(hidden_states, W_router, W1, b1, W2, b2):
    x = hidden_states.reshape(S, D)
    out = pl.pallas_call(
        _moe_dense_body,
        grid=(E, NF),
        in_specs=[
            pl.BlockSpec((S, D), lambda e, f: (0, 0)),
            pl.BlockSpec((D, E), lambda e, f: (0, 0)),
            pl.BlockSpec((1, D, FFB), lambda e, f: (e, 0, f)),
            pl.BlockSpec((1, FFB), lambda e, f: (e, f)),
            pl.BlockSpec((1, FFB, D), lambda e, f: (e, f, 0)),
            pl.BlockSpec((1, D), lambda e, f: (e, 0)),
        ],
        out_specs=pl.BlockSpec((S, D), lambda e, f: (0, 0)),
        out_shape=jax.ShapeDtypeStruct((S, D), jnp.float32),
        scratch_shapes=[pltpu.VMEM((S, E), jnp.float32)],
    )(x, W_router, W1, b1, W2, b2)
    return out.reshape(1, S, D)


# speedup vs baseline: 3.1159x; 3.1159x over previous
import jax
import jax.numpy as jnp
from jax.experimental import pallas as pl
from jax.experimental.pallas import tpu as pltpu

S, D, FF, E = 2048, 768, 3072, 8
FFB = 768
NF = FF // FFB
_SQRT2 = 1.4142135623730951


def _moe_dense_body(x_ref, wr_ref, w1_ref, b1_ref, w2_ref, b2_ref, out_ref,
                    gates_ref):
    e = pl.program_id(0)
    f = pl.program_id(1)

    @pl.when((e == 0) & (f == 0))
    def _init():
        logits = jnp.dot(x_ref[...], wr_ref[...],
                         preferred_element_type=jnp.float32)
        m = jnp.max(logits, axis=-1, keepdims=True)
        p = jnp.exp(logits - m)
        p = p / jnp.sum(p, axis=-1, keepdims=True)
        p1 = jnp.max(p, axis=-1, keepdims=True)
        is1 = p == p1
        pm = jnp.where(is1, -1.0, p)
        p2 = jnp.max(pm, axis=-1, keepdims=True)
        is2 = pm == p2
        denom = p1 + p2
        gates_ref[...] = jnp.where(is1, p1 / denom,
                                   jnp.where(is2, p2 / denom, 0.0))
        out_ref[...] = jnp.zeros_like(out_ref)

    lane = jax.lax.broadcasted_iota(jnp.int32, (S, E), 1)
    g = jnp.sum(jnp.where(lane == e, gates_ref[...], 0.0), axis=1,
                keepdims=True)
    h = jnp.dot(x_ref[...], w1_ref[0], preferred_element_type=jnp.float32)
    h = h + b1_ref[0]
    h = 0.5 * h * (1.0 + jax.lax.erf(h / _SQRT2))
    y = jnp.dot(h, w2_ref[0], preferred_element_type=jnp.float32)
    out_ref[...] += g * y

    @pl.when(f == 0)
    def _bias2():
        out_ref[...] += g * b2_ref[0]


def kernel(hidden_states, W_router, W1, b1, W2, b2):
    x = hidden_states.reshape(S, D)
    out = pl.pallas_call(
        _moe_dense_body,
        grid=(E, NF),
        in_specs=[
            pl.BlockSpec((S, D), lambda e, f: (0, 0)),
            pl.BlockSpec((D, E), lambda e, f: (0, 0)),
            pl.BlockSpec((1, D, FFB), lambda e, f: (e, 0, f)),
            pl.BlockSpec((1, 1, FFB), lambda e, f: (e, 0, f)),
            pl.BlockSpec((1, FFB, D), lambda e, f: (e, f, 0)),
            pl.BlockSpec((1, 1, D), lambda e, f: (e, 0, 0)),
        ],
        out_specs=pl.BlockSpec((S, D), lambda e, f: (0, 0)),
        out_shape=jax.ShapeDtypeStruct((S, D), jnp.float32),
        scratch_shapes=[pltpu.VMEM((S, E), jnp.float32)],
    )(x, W_router, W1, b1.reshape(E, 1, FF), W2, b2.reshape(E, 1, D))
    return out.reshape(1, S, D)
